# baseline (speedup 1.0000x reference)
import functools

import jax
import jax.numpy as jnp
from jax import lax
from jax.experimental import pallas as pl
from jax.experimental.pallas import tpu as pltpu
from jax.experimental.pallas import tpu_sc as plsc

_NC, _NS = 2, 16
_NW = _NC * _NS
_SLOPE = 0.01
_EPS = 1e-5
_GROUPS = 32


def _act(x):
    return jnp.where(x >= 0, x, _SLOPE * x)


def _mlp_stage(x, ws, residuals, block_rows, out_rows=None,
               out_dtype=jnp.float32, extra_bf16=False):
    m, c_in = x.shape
    c_out = ws[-1].shape[1]
    n_w = len(ws)
    m_out = m if out_rows is None else out_rows

    def body(*refs):
        h = refs[0][...].astype(jnp.float32)
        for i in range(n_w):
            h2 = _act(jnp.dot(h, refs[1 + i][...],
                              preferred_element_type=jnp.float32))
            h = h2 + h if residuals[i] else h2
        refs[n_w + 1][...] = h.astype(out_dtype)
        if extra_bf16:
            refs[n_w + 2][...] = h.astype(jnp.bfloat16)

    in_specs = [pl.BlockSpec((block_rows, c_in), lambda i: (i, 0))]
    for w in ws:
        in_specs.append(pl.BlockSpec(w.shape, lambda i: (0, 0)))
    out_specs = pl.BlockSpec((block_rows, c_out), lambda i: (i, 0))
    out_shape = jax.ShapeDtypeStruct((m_out, c_out), out_dtype)
    if extra_bf16:
        out_specs = (out_specs, pl.BlockSpec((block_rows, c_out),
                                             lambda i: (i, 0)))
        out_shape = (out_shape,
                     jax.ShapeDtypeStruct((m_out, c_out), jnp.bfloat16))
    return pl.pallas_call(
        body,
        grid=(m // block_rows,),
        in_specs=in_specs,
        out_specs=out_specs,
        out_shape=out_shape,
    )(x, *ws)


def _dec3_stage(g4, s3, wa, wb, bias, gamma, beta, pool_m, exp_m, block_rows,
                out_rows=None):
    m = g4.shape[0]
    c_out = wa.shape[1]
    m_out = m if out_rows is None else out_rows

    def body(g_ref, s_ref, wa_ref, wb_ref, b_ref, ga_ref, be_ref,
             p_ref, e_ref, o_ref):
        y = (jnp.dot(g_ref[...], wa_ref[...],
                     preferred_element_type=jnp.float32)
             + jnp.dot(s_ref[...], wb_ref[...],
                       preferred_element_type=jnp.float32)
             + b_ref[...])
        mu = jnp.dot(y, p_ref[...], preferred_element_type=jnp.float32)
        d = y - jnp.dot(mu, e_ref[...], preferred_element_type=jnp.float32)
        var = jnp.dot(d * d, p_ref[...], preferred_element_type=jnp.float32)
        inv = lax.rsqrt(var + _EPS)
        yn = d * jnp.dot(inv, e_ref[...], preferred_element_type=jnp.float32)
        o_ref[...] = _act(yn * ga_ref[...] + be_ref[...])

    specs = [
        pl.BlockSpec((block_rows, g4.shape[1]), lambda i: (i, 0)),
        pl.BlockSpec((block_rows, s3.shape[1]), lambda i: (i, 0)),
        pl.BlockSpec(wa.shape, lambda i: (0, 0)),
        pl.BlockSpec(wb.shape, lambda i: (0, 0)),
        pl.BlockSpec(bias.shape, lambda i: (0, 0)),
        pl.BlockSpec(gamma.shape, lambda i: (0, 0)),
        pl.BlockSpec(beta.shape, lambda i: (0, 0)),
        pl.BlockSpec(pool_m.shape, lambda i: (0, 0)),
        pl.BlockSpec(exp_m.shape, lambda i: (0, 0)),
    ]
    return pl.pallas_call(
        body,
        grid=(m // block_rows,),
        in_specs=specs,
        out_specs=pl.BlockSpec((block_rows, c_out), lambda i: (i, 0)),
        out_shape=jax.ShapeDtypeStruct((m_out, c_out), jnp.float32),
    )(g4, s3, wa, wb, bias, gamma, beta, pool_m, exp_m)


def _dec2_stage(g3, s2, wa, wb, block_rows, out_rows=None):
    m = g3.shape[0]
    c_out = wa.shape[1]
    m_out = m if out_rows is None else out_rows

    def body(g_ref, s_ref, wa_ref, wb_ref, o_ref):
        o_ref[...] = (jnp.dot(g_ref[...], wa_ref[...],
                              preferred_element_type=jnp.float32)
                      + jnp.dot(s_ref[...], wb_ref[...],
                                preferred_element_type=jnp.float32))

    specs = [
        pl.BlockSpec((block_rows, g3.shape[1]), lambda i: (i, 0)),
        pl.BlockSpec((block_rows, s2.shape[1]), lambda i: (i, 0)),
        pl.BlockSpec(wa.shape, lambda i: (0, 0)),
        pl.BlockSpec(wb.shape, lambda i: (0, 0)),
    ]
    return pl.pallas_call(
        body,
        grid=(m // block_rows,),
        in_specs=specs,
        out_specs=pl.BlockSpec((block_rows, c_out), lambda i: (i, 0)),
        out_shape=jax.ShapeDtypeStruct((m_out, c_out), jnp.float32),
    )(g3, s2, wa, wb)


def _sc_pool_colsplit(table, idx_flat, m_pad, k_nb, c_dim, r_blk, nbuf):
    c_half = c_dim // 2
    chunk = m_pad // _NS
    assert chunk % r_blk == 0
    nblk = chunk // r_blk
    n_idx = r_blk * k_nb
    assert n_idx % 8 == 0
    scale = 1.0 / k_nb
    nch = c_half // 16
    n_rows = table.shape[0]
    nsl = n_rows // _NS

    splits = []
    off = 0
    while off < n_idx:
        sz = min(128, n_idx - off)
        splits.append((off, sz))
        off += sz

    mesh = plsc.VectorSubcoreMesh(core_axis_name="c", subcore_axis_name="s")

    scratch = [pltpu.VMEM((n_idx,), jnp.int32) for _ in range(nbuf)]
    scratch += [pltpu.VMEM((n_idx, c_half), jnp.float32) for _ in range(nbuf)]
    scratch.append(pltpu.VMEM((r_blk, c_half), jnp.float32))
    scratch.append(pltpu.VMEM_SHARED((n_rows, c_half), jnp.float32))
    scratch += [pltpu.SemaphoreType.DMA for _ in range(nbuf)]

    def body(table_hbm, idx_hbm, out_hbm, *scr):
        idxb = scr[:nbuf]
        rowsb = scr[nbuf:2 * nbuf]
        out_v = scr[2 * nbuf]
        tbl_s = scr[2 * nbuf + 1]
        sems = scr[2 * nbuf + 2:]

        sid = lax.axis_index("s")
        cid = lax.axis_index("c")
        base = sid * chunk
        cb = cid * c_half

        pltpu.sync_copy(table_hbm.at[pl.ds(sid * nsl, nsl), pl.ds(cb, c_half)],
                        tbl_s.at[pl.ds(sid * nsl, nsl)])
        plsc.subcore_barrier()

        def issue(b, p):
            rb = base + b * r_blk
            pltpu.sync_copy(idx_hbm.at[pl.ds(rb * k_nb, n_idx)], idxb[p])
            for (o, s) in splits:
                pltpu.async_copy(tbl_s.at[idxb[p].at[pl.ds(o, s)]],
                                 rowsb[p].at[pl.ds(o, s)], sems[p])

        def drain(p):
            for (o, s) in splits:
                pltpu.make_async_copy(tbl_s.at[idxb[p].at[pl.ds(o, s)]],
                                      rowsb[p].at[pl.ds(o, s)],
                                      sems[p]).wait()

        def finish(b, p):
            rb = base + b * r_blk
            rows_v = rowsb[p]

            def row(r, rc):
                jb = r * k_nb
                accs = [None] * nch
                for kk in range(k_nb):
                    for ci in range(nch):
                        v = rows_v[jb + kk, pl.ds(ci * 16, 16)]
                        accs[ci] = v if kk == 0 else accs[ci] + v
                for ci in range(nch):
                    out_v[r, pl.ds(ci * 16, 16)] = accs[ci] * scale
                return rc

            lax.fori_loop(0, r_blk, row, 0)
            pltpu.sync_copy(out_v,
                            out_hbm.at[pl.ds(rb, r_blk), pl.ds(cb, c_half)])

        issue(0, 0)

        def blk(b, carry):
            @pl.when(b % 2 == 0)
            def _():
                pl.when(b + 1 < nblk)(lambda: issue(b + 1, 1))
                drain(0)
                finish(b, 0)

            @pl.when(b % 2 == 1)
            def _():
                pl.when(b + 1 < nblk)(lambda: issue(b + 1, 0))
                drain(1)
                finish(b, 1)

            return carry
        lax.fori_loop(0, nblk, blk, 0)

    f = pl.kernel(
        body,
        out_type=jax.ShapeDtypeStruct((m_pad, c_dim), jnp.float32),
        mesh=mesh,
        compiler_params=pltpu.CompilerParams(use_tc_tiling_on_sc=False),
        scratch_types=scratch,
    )
    return f(table, idx_flat)


def _sc_pool_bf16(table, idx_flat, m_pad, k_nb, c_dim, r_blk, nbuf,
                  col_split):
    cw = c_dim // 2 if col_split else c_dim
    chunk = m_pad // (_NS if col_split else _NW)
    assert chunk % r_blk == 0
    nblk = chunk // r_blk
    n_idx = r_blk * k_nb
    assert n_idx % 8 == 0
    scale = 1.0 / k_nb
    nch2 = cw // 32
    n_rows = table.shape[0]
    nsl = n_rows // _NS

    splits = []
    off = 0
    while off < n_idx:
        sz = min(128, n_idx - off)
        splits.append((off, sz))
        off += sz

    mesh = plsc.VectorSubcoreMesh(core_axis_name="c", subcore_axis_name="s")

    scratch = [pltpu.VMEM((n_idx,), jnp.int32) for _ in range(nbuf)]
    scratch += [pltpu.VMEM((n_idx, cw), jnp.bfloat16) for _ in range(nbuf)]
    scratch.append(pltpu.VMEM((r_blk, cw), jnp.float32))
    if col_split:
        scratch.append(pltpu.VMEM_SHARED((n_rows, cw), jnp.bfloat16))
    scratch += [pltpu.SemaphoreType.DMA for _ in range(nbuf)]

    def body(table_hbm, idx_hbm, out_hbm, *scr):
        pos = 0
        idxb = scr[pos:pos + nbuf]; pos += nbuf
        rowsb = scr[pos:pos + nbuf]; pos += nbuf
        out_v = scr[pos]; pos += 1
        tbl_s = None
        if col_split:
            tbl_s = scr[pos]; pos += 1
        sems = scr[pos:]

        sid = lax.axis_index("s")
        cid = lax.axis_index("c")
        if col_split:
            base = sid * chunk
            cb = cid * cw
            pltpu.sync_copy(
                table_hbm.at[pl.ds(sid * nsl, nsl), pl.ds(cb, cw)],
                tbl_s.at[pl.ds(sid * nsl, nsl)])
            plsc.subcore_barrier()
            src = tbl_s
        else:
            base = (sid * _NC + cid) * chunk
            cb = 0
            src = table_hbm

        def issue(b, p):
            rb = base + b * r_blk
            pltpu.sync_copy(idx_hbm.at[pl.ds(rb * k_nb, n_idx)], idxb[p])
            for (o, s) in splits:
                pltpu.async_copy(src.at[idxb[p].at[pl.ds(o, s)]],
                                 rowsb[p].at[pl.ds(o, s)], sems[p])

        def drain(p):
            for (o, s) in splits:
                pltpu.make_async_copy(src.at[idxb[p].at[pl.ds(o, s)]],
                                      rowsb[p].at[pl.ds(o, s)],
                                      sems[p]).wait()

        def finish(b, p):
            rb = base + b * r_blk
            rows_v = rowsb[p]

            def row(r, rc):
                jb = r * k_nb
                accs = [None] * (2 * nch2)
                for kk in range(k_nb):
                    for ci in range(nch2):
                        vb = rows_v[jb + kk, pl.ds(ci * 32, 32)]
                        a, b2 = plsc.unpack(
                            vb, format=plsc.PackFormat.INTERLEAVED,
                            preferred_element_type=jnp.float32)
                        if kk == 0:
                            accs[2 * ci], accs[2 * ci + 1] = a, b2
                        else:
                            accs[2 * ci] = accs[2 * ci] + a
                            accs[2 * ci + 1] = accs[2 * ci + 1] + b2
                for ci in range(2 * nch2):
                    out_v[r, pl.ds(ci * 16, 16)] = accs[ci] * scale
                return rc

            lax.fori_loop(0, r_blk, row, 0)
            if col_split:
                pltpu.sync_copy(
                    out_v, out_hbm.at[pl.ds(rb, r_blk), pl.ds(cb, cw)])
            else:
                pltpu.sync_copy(out_v, out_hbm.at[pl.ds(rb, r_blk)])

        issue(0, 0)

        def blk(b, carry):
            @pl.when(b % 2 == 0)
            def _():
                pl.when(b + 1 < nblk)(lambda: issue(b + 1, 1))
                drain(0)
                finish(b, 0)

            @pl.when(b % 2 == 1)
            def _():
                pl.when(b + 1 < nblk)(lambda: issue(b + 1, 0))
                drain(1)
                finish(b, 1)

            return carry
        lax.fori_loop(0, nblk, blk, 0)

    f = pl.kernel(
        body,
        out_type=jax.ShapeDtypeStruct((m_pad, c_dim), jnp.float32),
        mesh=mesh,
        compiler_params=pltpu.CompilerParams(use_tc_tiling_on_sc=False,
                                             needs_layout_passes=False),
        scratch_types=scratch,
    )
    return f(table, idx_flat)


def _deint_perm(c_dim):
    p = jnp.arange(c_dim, dtype=jnp.int32)
    chunk = (p // 32) * 32
    pos = p % 32
    return jnp.where(pos < 16, chunk + 2 * pos, chunk + 2 * (pos - 16) + 1)


def _sc_pool(table, idx_flat, m_pad, k_nb, c_dim, r_blk, nbuf,
             use_spmem=False, tc_tiling=False):
    chunk = m_pad // _NW
    assert chunk % r_blk == 0 and chunk % 8 == 0
    nblk = chunk // r_blk
    n_idx = r_blk * k_nb
    assert n_idx % 8 == 0
    scale = 1.0 / k_nb
    nch = c_dim // 16

    splits = []
    off = 0
    while off < n_idx:
        sz = min(128, n_idx - off)
        splits.append((off, sz))
        off += sz

    mesh = plsc.VectorSubcoreMesh(core_axis_name="c", subcore_axis_name="s")

    n_rows = table.shape[0]
    scratch = [pltpu.VMEM((n_idx,), jnp.int32) for _ in range(nbuf)]
    scratch += [pltpu.VMEM((n_idx, c_dim), jnp.float32) for _ in range(nbuf)]
    if k_nb > 1:
        scratch.append(pltpu.VMEM((r_blk, c_dim), jnp.float32))
    if use_spmem:
        scratch.append(pltpu.VMEM_SHARED((n_rows, c_dim), jnp.float32))
    scratch += [pltpu.SemaphoreType.DMA for _ in range(nbuf)]

    def body(table_hbm, idx_hbm, out_hbm, *scr):
        pos = 0
        idxb = scr[pos:pos + nbuf]; pos += nbuf
        rowsb = scr[pos:pos + nbuf]; pos += nbuf
        out_v = None
        if k_nb > 1:
            out_v = scr[pos]; pos += 1
        tbl_s = None
        if use_spmem:
            tbl_s = scr[pos]; pos += 1
        sems = scr[pos:]

        wid = lax.axis_index("s") * _NC + lax.axis_index("c")
        base = wid * chunk

        if use_spmem:
            nsl = n_rows // _NS
            sid = lax.axis_index("s")
            pltpu.sync_copy(table_hbm.at[pl.ds(sid * nsl, nsl)],
                            tbl_s.at[pl.ds(sid * nsl, nsl)])
            plsc.subcore_barrier()
            src = tbl_s
        else:
            src = table_hbm

        def issue(b, p):
            rb = base + b * r_blk
            pltpu.sync_copy(idx_hbm.at[pl.ds(rb * k_nb, n_idx)], idxb[p])
            for (o, s) in splits:
                pltpu.async_copy(src.at[idxb[p].at[pl.ds(o, s)]],
                                 rowsb[p].at[pl.ds(o, s)], sems[p])

        def drain(p):
            for (o, s) in splits:
                pltpu.make_async_copy(src.at[idxb[p].at[pl.ds(o, s)]],
                                      rowsb[p].at[pl.ds(o, s)],
                                      sems[p]).wait()

        def finish(b, p):
            rb = base + b * r_blk
            rows_v = rowsb[p]
            if k_nb == 1:
                pltpu.sync_copy(rows_v, out_hbm.at[pl.ds(rb, r_blk)])
                return

            def row(r, rc):
                jb = r * k_nb
                accs = [None] * nch
                for kk in range(k_nb):
                    for ci in range(nch):
                        v = rows_v[jb + kk, pl.ds(ci * 16, 16)]
                        accs[ci] = v if kk == 0 else accs[ci] + v
                for ci in range(nch):
                    out_v[r, pl.ds(ci * 16, 16)] = accs[ci] * scale
                return rc

            lax.fori_loop(0, r_blk, row, 0)
            pltpu.sync_copy(out_v, out_hbm.at[pl.ds(rb, r_blk)])

        if nbuf == 1:
            def blk(b, carry):
                issue(b, 0)
                drain(0)
                finish(b, 0)
                return carry
            lax.fori_loop(0, nblk, blk, 0)
        else:
            issue(0, 0)

            def blk(b, carry):
                @pl.when(b % 2 == 0)
                def _():
                    pl.when(b + 1 < nblk)(lambda: issue(b + 1, 1))
                    drain(0)
                    finish(b, 0)

                @pl.when(b % 2 == 1)
                def _():
                    pl.when(b + 1 < nblk)(lambda: issue(b + 1, 0))
                    drain(1)
                    finish(b, 1)

                return carry
            lax.fori_loop(0, nblk, blk, 0)

    f = pl.kernel(
        body,
        out_type=jax.ShapeDtypeStruct((m_pad, c_dim), jnp.float32),
        mesh=mesh,
        compiler_params=pltpu.CompilerParams(use_tc_tiling_on_sc=tc_tiling),
        scratch_types=scratch,
    )
    return f(table, idx_flat)


def _pad_rows(a, m_pad):
    return jnp.pad(a, ((0, m_pad - a.shape[0]),) + ((0, 0),) * (a.ndim - 1))


def kernel(feats, points_s1, points_s2, points_s3, points_s4,
           W0, W1, W2, W3, W4, W5, W6, W7, W8, W9, W10,
           W_d3, b_d3, gamma_d3, beta_d3, W_d2,
           subsampling_0, subsampling_1, subsampling_2,
           upsampling_1, upsampling_2):
    m2p, m3p, m4p = 12544, 3328, 1024

    sub0 = _pad_rows(subsampling_0, m2p).reshape(-1)
    sub1 = _pad_rows(subsampling_1, m3p).reshape(-1)
    sub2 = _pad_rows(subsampling_2, m4p).reshape(-1)
    up1 = _pad_rows(upsampling_1, m2p).reshape(-1)
    up2 = _pad_rows(upsampling_2, m3p).reshape(-1)

    ch = jnp.arange(256, dtype=jnp.int32)
    gr = jnp.arange(_GROUPS, dtype=jnp.int32)
    pool_m = ((ch[:, None] // 8) == gr[None, :]).astype(jnp.float32) / 8.0
    exp_m = ((gr[:, None] == ch[None, :] // 8)).astype(jnp.float32)

    s1 = _mlp_stage(feats, [W0, W1], [False, False], 2000,
                    out_dtype=jnp.bfloat16)
    p0 = _sc_pool_bf16(s1, sub0, m2p, 26, 64, 56, 2,
                       col_split=False)
    s2, s2b = _mlp_stage(p0, [W2[_deint_perm(64)], W3, W4],
                         [False, False, True], 1568,
                         extra_bf16=True)
    p1 = _sc_pool_bf16(s2b, sub1, m3p, 31, 128, 16, 2,
                       col_split=True)
    s3, s3b = _mlp_stage(p1, [W5[_deint_perm(128)], W6, W7],
                         [False, False, True], 832,
                         extra_bf16=True)
    p2 = _sc_pool_bf16(s3b, sub2, m4p, 33, 256, 8, 2,
                       col_split=True)
    s4 = _mlp_stage(p2, [W8[_deint_perm(256)], W9, W10],
                    [False, False, True], 1024,
                    out_rows=782)

    g4 = _sc_pool(s4, up2, m3p, 1, 512, 8, 2, tc_tiling=True)
    l3 = _dec3_stage(g4, s3, W_d3[:512], W_d3[512:],
                     b_d3.reshape(1, -1), gamma_d3.reshape(1, -1),
                     beta_d3.reshape(1, -1), pool_m, exp_m, 832,
                     out_rows=3125)
    g3 = _sc_pool(l3, up1, m2p, 1, 256, 56, 2, tc_tiling=True)
    l2 = _dec2_stage(g3, s2, W_d2[:256], W_d2[256:], 1568,
                     out_rows=12500)

    return (l2, l3, s4)


# speedup vs baseline: 1.0323x; 1.0323x over previous
import functools

import jax
import jax.numpy as jnp
from jax import lax
from jax.experimental import pallas as pl
from jax.experimental.pallas import tpu as pltpu
from jax.experimental.pallas import tpu_sc as plsc

_NC, _NS = 2, 16
_NW = _NC * _NS
_SLOPE = 0.01
_EPS = 1e-5
_GROUPS = 32


def _act(x):
    return jnp.where(x >= 0, x, _SLOPE * x)


def _mlp_stage(x, ws, residuals, block_rows, out_rows=None,
               out_dtype=jnp.float32, extra_bf16=False):
    m, c_in = x.shape
    c_out = ws[-1].shape[1]
    n_w = len(ws)
    m_out = m if out_rows is None else out_rows

    def body(*refs):
        h = refs[0][...].astype(jnp.float32)
        for i in range(n_w):
            h2 = _act(jnp.dot(h, refs[1 + i][...],
                              preferred_element_type=jnp.float32))
            h = h2 + h if residuals[i] else h2
        refs[n_w + 1][...] = h.astype(out_dtype)
        if extra_bf16:
            refs[n_w + 2][...] = h.astype(jnp.bfloat16)

    in_specs = [pl.BlockSpec((block_rows, c_in), lambda i: (i, 0))]
    for w in ws:
        in_specs.append(pl.BlockSpec(w.shape, lambda i: (0, 0)))
    out_specs = pl.BlockSpec((block_rows, c_out), lambda i: (i, 0))
    out_shape = jax.ShapeDtypeStruct((m_out, c_out), out_dtype)
    if extra_bf16:
        out_specs = (out_specs, pl.BlockSpec((block_rows, c_out),
                                             lambda i: (i, 0)))
        out_shape = (out_shape,
                     jax.ShapeDtypeStruct((m_out, c_out), jnp.bfloat16))
    return pl.pallas_call(
        body,
        grid=(m // block_rows,),
        in_specs=in_specs,
        out_specs=out_specs,
        out_shape=out_shape,
    )(x, *ws)


def _dec3_stage(g4, s3, wa, wb, bias, gamma, beta, pool_m, exp_m, block_rows,
                out_rows=None):
    m = g4.shape[0]
    c_out = wa.shape[1]
    m_out = m if out_rows is None else out_rows

    def body(g_ref, s_ref, wa_ref, wb_ref, b_ref, ga_ref, be_ref,
             p_ref, e_ref, o_ref):
        y = (jnp.dot(g_ref[...], wa_ref[...],
                     preferred_element_type=jnp.float32)
             + jnp.dot(s_ref[...], wb_ref[...],
                       preferred_element_type=jnp.float32)
             + b_ref[...])
        mu = jnp.dot(y, p_ref[...], preferred_element_type=jnp.float32)
        d = y - jnp.dot(mu, e_ref[...], preferred_element_type=jnp.float32)
        var = jnp.dot(d * d, p_ref[...], preferred_element_type=jnp.float32)
        inv = lax.rsqrt(var + _EPS)
        yn = d * jnp.dot(inv, e_ref[...], preferred_element_type=jnp.float32)
        o_ref[...] = _act(yn * ga_ref[...] + be_ref[...])

    specs = [
        pl.BlockSpec((block_rows, g4.shape[1]), lambda i: (i, 0)),
        pl.BlockSpec((block_rows, s3.shape[1]), lambda i: (i, 0)),
        pl.BlockSpec(wa.shape, lambda i: (0, 0)),
        pl.BlockSpec(wb.shape, lambda i: (0, 0)),
        pl.BlockSpec(bias.shape, lambda i: (0, 0)),
        pl.BlockSpec(gamma.shape, lambda i: (0, 0)),
        pl.BlockSpec(beta.shape, lambda i: (0, 0)),
        pl.BlockSpec(pool_m.shape, lambda i: (0, 0)),
        pl.BlockSpec(exp_m.shape, lambda i: (0, 0)),
    ]
    return pl.pallas_call(
        body,
        grid=(m // block_rows,),
        in_specs=specs,
        out_specs=pl.BlockSpec((block_rows, c_out), lambda i: (i, 0)),
        out_shape=jax.ShapeDtypeStruct((m_out, c_out), jnp.float32),
    )(g4, s3, wa, wb, bias, gamma, beta, pool_m, exp_m)


def _dec2_stage(g3, s2, wa, wb, block_rows, out_rows=None):
    m = g3.shape[0]
    c_out = wa.shape[1]
    m_out = m if out_rows is None else out_rows

    def body(g_ref, s_ref, wa_ref, wb_ref, o_ref):
        o_ref[...] = (jnp.dot(g_ref[...], wa_ref[...],
                              preferred_element_type=jnp.float32)
                      + jnp.dot(s_ref[...], wb_ref[...],
                                preferred_element_type=jnp.float32))

    specs = [
        pl.BlockSpec((block_rows, g3.shape[1]), lambda i: (i, 0)),
        pl.BlockSpec((block_rows, s2.shape[1]), lambda i: (i, 0)),
        pl.BlockSpec(wa.shape, lambda i: (0, 0)),
        pl.BlockSpec(wb.shape, lambda i: (0, 0)),
    ]
    return pl.pallas_call(
        body,
        grid=(m // block_rows,),
        in_specs=specs,
        out_specs=pl.BlockSpec((block_rows, c_out), lambda i: (i, 0)),
        out_shape=jax.ShapeDtypeStruct((m_out, c_out), jnp.float32),
    )(g3, s2, wa, wb)


def _sc_pool_colsplit(table, idx_flat, m_pad, k_nb, c_dim, r_blk, nbuf):
    c_half = c_dim // 2
    chunk = m_pad // _NS
    assert chunk % r_blk == 0
    nblk = chunk // r_blk
    n_idx = r_blk * k_nb
    assert n_idx % 8 == 0
    scale = 1.0 / k_nb
    nch = c_half // 16
    n_rows = table.shape[0]
    nsl = n_rows // _NS

    splits = []
    off = 0
    while off < n_idx:
        sz = min(128, n_idx - off)
        splits.append((off, sz))
        off += sz

    mesh = plsc.VectorSubcoreMesh(core_axis_name="c", subcore_axis_name="s")

    scratch = [pltpu.VMEM((n_idx,), jnp.int32) for _ in range(nbuf)]
    scratch += [pltpu.VMEM((n_idx, c_half), jnp.float32) for _ in range(nbuf)]
    scratch.append(pltpu.VMEM((r_blk, c_half), jnp.float32))
    scratch.append(pltpu.VMEM_SHARED((n_rows, c_half), jnp.float32))
    scratch += [pltpu.SemaphoreType.DMA for _ in range(nbuf)]

    def body(table_hbm, idx_hbm, out_hbm, *scr):
        idxb = scr[:nbuf]
        rowsb = scr[nbuf:2 * nbuf]
        out_v = scr[2 * nbuf]
        tbl_s = scr[2 * nbuf + 1]
        sems = scr[2 * nbuf + 2:]

        sid = lax.axis_index("s")
        cid = lax.axis_index("c")
        base = sid * chunk
        cb = cid * c_half

        pltpu.sync_copy(table_hbm.at[pl.ds(sid * nsl, nsl), pl.ds(cb, c_half)],
                        tbl_s.at[pl.ds(sid * nsl, nsl)])
        plsc.subcore_barrier()

        def issue(b, p):
            rb = base + b * r_blk
            pltpu.sync_copy(idx_hbm.at[pl.ds(rb * k_nb, n_idx)], idxb[p])
            for (o, s) in splits:
                pltpu.async_copy(tbl_s.at[idxb[p].at[pl.ds(o, s)]],
                                 rowsb[p].at[pl.ds(o, s)], sems[p])

        def drain(p):
            for (o, s) in splits:
                pltpu.make_async_copy(tbl_s.at[idxb[p].at[pl.ds(o, s)]],
                                      rowsb[p].at[pl.ds(o, s)],
                                      sems[p]).wait()

        def finish(b, p):
            rb = base + b * r_blk
            rows_v = rowsb[p]

            def row(r, rc):
                jb = r * k_nb
                accs = [None] * nch
                for kk in range(k_nb):
                    for ci in range(nch):
                        v = rows_v[jb + kk, pl.ds(ci * 16, 16)]
                        accs[ci] = v if kk == 0 else accs[ci] + v
                for ci in range(nch):
                    out_v[r, pl.ds(ci * 16, 16)] = accs[ci] * scale
                return rc

            lax.fori_loop(0, r_blk, row, 0)
            pltpu.sync_copy(out_v,
                            out_hbm.at[pl.ds(rb, r_blk), pl.ds(cb, c_half)])

        issue(0, 0)

        def blk(b, carry):
            @pl.when(b % 2 == 0)
            def _():
                pl.when(b + 1 < nblk)(lambda: issue(b + 1, 1))
                drain(0)
                finish(b, 0)

            @pl.when(b % 2 == 1)
            def _():
                pl.when(b + 1 < nblk)(lambda: issue(b + 1, 0))
                drain(1)
                finish(b, 1)

            return carry
        lax.fori_loop(0, nblk, blk, 0)

    f = pl.kernel(
        body,
        out_type=jax.ShapeDtypeStruct((m_pad, c_dim), jnp.float32),
        mesh=mesh,
        compiler_params=pltpu.CompilerParams(use_tc_tiling_on_sc=False),
        scratch_types=scratch,
    )
    return f(table, idx_flat)


def _sc_pool_bf16(table, idx_flat, m_pad, k_nb, c_dim, r_blk, nbuf,
                  col_split):
    cw = c_dim // 2 if col_split else c_dim
    chunk = m_pad // (_NS if col_split else _NW)
    assert chunk % r_blk == 0
    nblk = chunk // r_blk
    n_idx = r_blk * k_nb
    assert n_idx % 8 == 0
    scale = 1.0 / k_nb
    nch2 = cw // 32
    n_rows = table.shape[0]
    nsl = n_rows // _NS

    splits = []
    off = 0
    while off < n_idx:
        sz = min(128, n_idx - off)
        splits.append((off, sz))
        off += sz

    mesh = plsc.VectorSubcoreMesh(core_axis_name="c", subcore_axis_name="s")

    scratch = [pltpu.VMEM((chunk * k_nb,), jnp.int32)]
    scratch += [pltpu.VMEM((n_idx, cw), jnp.bfloat16) for _ in range(nbuf)]
    scratch.append(pltpu.VMEM((r_blk, cw), jnp.float32))
    if col_split:
        scratch.append(pltpu.VMEM_SHARED((n_rows, cw), jnp.bfloat16))
    scratch += [pltpu.SemaphoreType.DMA for _ in range(nbuf)]

    def body(table_hbm, idx_hbm, out_hbm, *scr):
        pos = 0
        idx_v = scr[pos]; pos += 1
        rowsb = scr[pos:pos + nbuf]; pos += nbuf
        out_v = scr[pos]; pos += 1
        tbl_s = None
        if col_split:
            tbl_s = scr[pos]; pos += 1
        sems = scr[pos:]

        sid = lax.axis_index("s")
        cid = lax.axis_index("c")
        if col_split:
            base = sid * chunk
            cb = cid * cw
            pltpu.sync_copy(idx_hbm.at[pl.ds(base * k_nb, chunk * k_nb)],
                            idx_v)
            pltpu.sync_copy(
                table_hbm.at[pl.ds(sid * nsl, nsl), pl.ds(cb, cw)],
                tbl_s.at[pl.ds(sid * nsl, nsl)])
            plsc.subcore_barrier()
            src = tbl_s
        else:
            base = (sid * _NC + cid) * chunk
            cb = 0
            pltpu.sync_copy(idx_hbm.at[pl.ds(base * k_nb, chunk * k_nb)],
                            idx_v)
            src = table_hbm

        def issue(b, p):
            for (o, s) in splits:
                pltpu.async_copy(
                    src.at[idx_v.at[pl.ds(b * n_idx + o, s)]],
                    rowsb[p].at[pl.ds(o, s)], sems[p])

        def drain(b, p):
            for (o, s) in splits:
                pltpu.make_async_copy(
                    src.at[idx_v.at[pl.ds(b * n_idx + o, s)]],
                    rowsb[p].at[pl.ds(o, s)], sems[p]).wait()

        def finish(b, p):
            rb = base + b * r_blk
            rows_v = rowsb[p]

            def row(r, rc):
                jb = r * k_nb
                accs = [None] * (2 * nch2)
                for kk in range(k_nb):
                    for ci in range(nch2):
                        vb = rows_v[jb + kk, pl.ds(ci * 32, 32)]
                        a, b2 = plsc.unpack(
                            vb, format=plsc.PackFormat.INTERLEAVED,
                            preferred_element_type=jnp.float32)
                        if kk == 0:
                            accs[2 * ci], accs[2 * ci + 1] = a, b2
                        else:
                            accs[2 * ci] = accs[2 * ci] + a
                            accs[2 * ci + 1] = accs[2 * ci + 1] + b2
                for ci in range(2 * nch2):
                    out_v[r, pl.ds(ci * 16, 16)] = accs[ci] * scale
                return rc

            lax.fori_loop(0, r_blk, row, 0)
            if col_split:
                pltpu.sync_copy(
                    out_v, out_hbm.at[pl.ds(rb, r_blk), pl.ds(cb, cw)])
            else:
                pltpu.sync_copy(out_v, out_hbm.at[pl.ds(rb, r_blk)])

        issue(0, 0)

        def blk(b, carry):
            @pl.when(b % 2 == 0)
            def _():
                pl.when(b + 1 < nblk)(lambda: issue(b + 1, 1))
                drain(b, 0)
                finish(b, 0)

            @pl.when(b % 2 == 1)
            def _():
                pl.when(b + 1 < nblk)(lambda: issue(b + 1, 0))
                drain(b, 1)
                finish(b, 1)

            return carry
        lax.fori_loop(0, nblk, blk, 0)

    f = pl.kernel(
        body,
        out_type=jax.ShapeDtypeStruct((m_pad, c_dim), jnp.float32),
        mesh=mesh,
        compiler_params=pltpu.CompilerParams(use_tc_tiling_on_sc=False,
                                             needs_layout_passes=False),
        scratch_types=scratch,
    )
    return f(table, idx_flat)


def _deint_perm(c_dim):
    p = jnp.arange(c_dim, dtype=jnp.int32)
    chunk = (p // 32) * 32
    pos = p % 32
    return jnp.where(pos < 16, chunk + 2 * pos, chunk + 2 * (pos - 16) + 1)


def _sc_pool(table, idx_flat, m_pad, k_nb, c_dim, r_blk, nbuf,
             use_spmem=False, tc_tiling=False):
    chunk = m_pad // _NW
    assert chunk % r_blk == 0 and chunk % 8 == 0
    nblk = chunk // r_blk
    n_idx = r_blk * k_nb
    assert n_idx % 8 == 0
    scale = 1.0 / k_nb
    nch = c_dim // 16

    splits = []
    off = 0
    while off < n_idx:
        sz = min(128, n_idx - off)
        splits.append((off, sz))
        off += sz

    mesh = plsc.VectorSubcoreMesh(core_axis_name="c", subcore_axis_name="s")

    n_rows = table.shape[0]
    scratch = [pltpu.VMEM((n_idx,), jnp.int32) for _ in range(nbuf)]
    scratch += [pltpu.VMEM((n_idx, c_dim), jnp.float32) for _ in range(nbuf)]
    if k_nb > 1:
        scratch.append(pltpu.VMEM((r_blk, c_dim), jnp.float32))
    if use_spmem:
        scratch.append(pltpu.VMEM_SHARED((n_rows, c_dim), jnp.float32))
    scratch += [pltpu.SemaphoreType.DMA for _ in range(nbuf)]

    def body(table_hbm, idx_hbm, out_hbm, *scr):
        pos = 0
        idxb = scr[pos:pos + nbuf]; pos += nbuf
        rowsb = scr[pos:pos + nbuf]; pos += nbuf
        out_v = None
        if k_nb > 1:
            out_v = scr[pos]; pos += 1
        tbl_s = None
        if use_spmem:
            tbl_s = scr[pos]; pos += 1
        sems = scr[pos:]

        wid = lax.axis_index("s") * _NC + lax.axis_index("c")
        base = wid * chunk

        if use_spmem:
            nsl = n_rows // _NS
            sid = lax.axis_index("s")
            pltpu.sync_copy(table_hbm.at[pl.ds(sid * nsl, nsl)],
                            tbl_s.at[pl.ds(sid * nsl, nsl)])
            plsc.subcore_barrier()
            src = tbl_s
        else:
            src = table_hbm

        def issue(b, p):
            rb = base + b * r_blk
            pltpu.sync_copy(idx_hbm.at[pl.ds(rb * k_nb, n_idx)], idxb[p])
            for (o, s) in splits:
                pltpu.async_copy(src.at[idxb[p].at[pl.ds(o, s)]],
                                 rowsb[p].at[pl.ds(o, s)], sems[p])

        def drain(p):
            for (o, s) in splits:
                pltpu.make_async_copy(src.at[idxb[p].at[pl.ds(o, s)]],
                                      rowsb[p].at[pl.ds(o, s)],
                                      sems[p]).wait()

        def finish(b, p):
            rb = base + b * r_blk
            rows_v = rowsb[p]
            if k_nb == 1:
                pltpu.sync_copy(rows_v, out_hbm.at[pl.ds(rb, r_blk)])
                return

            def row(r, rc):
                jb = r * k_nb
                accs = [None] * nch
                for kk in range(k_nb):
                    for ci in range(nch):
                        v = rows_v[jb + kk, pl.ds(ci * 16, 16)]
                        accs[ci] = v if kk == 0 else accs[ci] + v
                for ci in range(nch):
                    out_v[r, pl.ds(ci * 16, 16)] = accs[ci] * scale
                return rc

            lax.fori_loop(0, r_blk, row, 0)
            pltpu.sync_copy(out_v, out_hbm.at[pl.ds(rb, r_blk)])

        if nbuf == 1:
            def blk(b, carry):
                issue(b, 0)
                drain(0)
                finish(b, 0)
                return carry
            lax.fori_loop(0, nblk, blk, 0)
        else:
            issue(0, 0)

            def blk(b, carry):
                @pl.when(b % 2 == 0)
                def _():
                    pl.when(b + 1 < nblk)(lambda: issue(b + 1, 1))
                    drain(0)
                    finish(b, 0)

                @pl.when(b % 2 == 1)
                def _():
                    pl.when(b + 1 < nblk)(lambda: issue(b + 1, 0))
                    drain(1)
                    finish(b, 1)

                return carry
            lax.fori_loop(0, nblk, blk, 0)

    f = pl.kernel(
        body,
        out_type=jax.ShapeDtypeStruct((m_pad, c_dim), jnp.float32),
        mesh=mesh,
        compiler_params=pltpu.CompilerParams(use_tc_tiling_on_sc=tc_tiling),
        scratch_types=scratch,
    )
    return f(table, idx_flat)


def _pad_rows(a, m_pad):
    return jnp.pad(a, ((0, m_pad - a.shape[0]),) + ((0, 0),) * (a.ndim - 1))


def kernel(feats, points_s1, points_s2, points_s3, points_s4,
           W0, W1, W2, W3, W4, W5, W6, W7, W8, W9, W10,
           W_d3, b_d3, gamma_d3, beta_d3, W_d2,
           subsampling_0, subsampling_1, subsampling_2,
           upsampling_1, upsampling_2):
    m2p, m3p, m4p = 12544, 3328, 1024

    sub0 = _pad_rows(subsampling_0, m2p).reshape(-1)
    sub1 = _pad_rows(subsampling_1, m3p).reshape(-1)
    sub2 = _pad_rows(subsampling_2, m4p).reshape(-1)
    up1 = _pad_rows(upsampling_1, m2p).reshape(-1)
    up2 = _pad_rows(upsampling_2, m3p).reshape(-1)

    ch = jnp.arange(256, dtype=jnp.int32)
    gr = jnp.arange(_GROUPS, dtype=jnp.int32)
    pool_m = ((ch[:, None] // 8) == gr[None, :]).astype(jnp.float32) / 8.0
    exp_m = ((gr[:, None] == ch[None, :] // 8)).astype(jnp.float32)

    s1 = _mlp_stage(feats, [W0, W1], [False, False], 2000,
                    out_dtype=jnp.bfloat16)
    p0 = _sc_pool_bf16(s1, sub0, m2p, 26, 64, 56, 2,
                       col_split=False)
    s2, s2b = _mlp_stage(p0, [W2[_deint_perm(64)], W3, W4],
                         [False, False, True], 1568,
                         extra_bf16=True)
    p1 = _sc_pool_bf16(s2b, sub1, m3p, 31, 128, 16, 2,
                       col_split=True)
    s3, s3b = _mlp_stage(p1, [W5[_deint_perm(128)], W6, W7],
                         [False, False, True], 832,
                         extra_bf16=True)
    p2 = _sc_pool_bf16(s3b, sub2, m4p, 33, 256, 8, 2,
                       col_split=True)
    s4 = _mlp_stage(p2, [W8[_deint_perm(256)], W9, W10],
                    [False, False, True], 1024,
                    out_rows=782)

    g4 = _sc_pool(s4, up2, m3p, 1, 512, 104, 1, tc_tiling=True)
    l3 = _dec3_stage(g4, s3, W_d3[:512], W_d3[512:],
                     b_d3.reshape(1, -1), gamma_d3.reshape(1, -1),
                     beta_d3.reshape(1, -1), pool_m, exp_m, 832,
                     out_rows=3125)
    g3 = _sc_pool(l3, up1, m2p, 1, 256, 392, 1, tc_tiling=True)
    l2 = _dec2_stage(g3, s2, W_d2[:256], W_d2[256:], 1568,
                     out_rows=12500)

    return (l2, l3, s4)


# speedup vs baseline: 1.0366x; 1.0041x over previous
import functools

import jax
import jax.numpy as jnp
from jax import lax
from jax.experimental import pallas as pl
from jax.experimental.pallas import tpu as pltpu
from jax.experimental.pallas import tpu_sc as plsc

_NC, _NS = 2, 16
_NW = _NC * _NS
_SLOPE = 0.01
_EPS = 1e-5
_GROUPS = 32


def _act(x):
    return jnp.where(x >= 0, x, _SLOPE * x)


def _mlp_stage(x, ws, residuals, block_rows, out_rows=None,
               out_dtype=jnp.float32, extra_bf16=False):
    m, c_in = x.shape
    c_out = ws[-1].shape[1]
    n_w = len(ws)
    m_out = m if out_rows is None else out_rows

    def body(*refs):
        h = refs[0][...].astype(jnp.float32)
        for i in range(n_w):
            h2 = _act(jnp.dot(h, refs[1 + i][...],
                              preferred_element_type=jnp.float32))
            h = h2 + h if residuals[i] else h2
        refs[n_w + 1][...] = h.astype(out_dtype)
        if extra_bf16:
            refs[n_w + 2][...] = h.astype(jnp.bfloat16)

    in_specs = [pl.BlockSpec((block_rows, c_in), lambda i: (i, 0))]
    for w in ws:
        in_specs.append(pl.BlockSpec(w.shape, lambda i: (0, 0)))
    out_specs = pl.BlockSpec((block_rows, c_out), lambda i: (i, 0))
    out_shape = jax.ShapeDtypeStruct((m_out, c_out), out_dtype)
    if extra_bf16:
        out_specs = (out_specs, pl.BlockSpec((block_rows, c_out),
                                             lambda i: (i, 0)))
        out_shape = (out_shape,
                     jax.ShapeDtypeStruct((m_out, c_out), jnp.bfloat16))
    return pl.pallas_call(
        body,
        grid=(m // block_rows,),
        in_specs=in_specs,
        out_specs=out_specs,
        out_shape=out_shape,
    )(x, *ws)


def _dec3_stage(g4, s3, wa, wb, bias, gamma, beta, pool_m, exp_m, block_rows,
                out_rows=None):
    m = g4.shape[0]
    c_out = wa.shape[1]
    m_out = m if out_rows is None else out_rows

    def body(g_ref, s_ref, wa_ref, wb_ref, b_ref, ga_ref, be_ref,
             p_ref, e_ref, o_ref):
        y = (jnp.dot(g_ref[...], wa_ref[...],
                     preferred_element_type=jnp.float32)
             + jnp.dot(s_ref[...], wb_ref[...],
                       preferred_element_type=jnp.float32)
             + b_ref[...])
        mu = jnp.dot(y, p_ref[...], preferred_element_type=jnp.float32)
        d = y - jnp.dot(mu, e_ref[...], preferred_element_type=jnp.float32)
        var = jnp.dot(d * d, p_ref[...], preferred_element_type=jnp.float32)
        inv = lax.rsqrt(var + _EPS)
        yn = d * jnp.dot(inv, e_ref[...], preferred_element_type=jnp.float32)
        o_ref[...] = _act(yn * ga_ref[...] + be_ref[...])

    specs = [
        pl.BlockSpec((block_rows, g4.shape[1]), lambda i: (i, 0)),
        pl.BlockSpec((block_rows, s3.shape[1]), lambda i: (i, 0)),
        pl.BlockSpec(wa.shape, lambda i: (0, 0)),
        pl.BlockSpec(wb.shape, lambda i: (0, 0)),
        pl.BlockSpec(bias.shape, lambda i: (0, 0)),
        pl.BlockSpec(gamma.shape, lambda i: (0, 0)),
        pl.BlockSpec(beta.shape, lambda i: (0, 0)),
        pl.BlockSpec(pool_m.shape, lambda i: (0, 0)),
        pl.BlockSpec(exp_m.shape, lambda i: (0, 0)),
    ]
    return pl.pallas_call(
        body,
        grid=(m // block_rows,),
        in_specs=specs,
        out_specs=pl.BlockSpec((block_rows, c_out), lambda i: (i, 0)),
        out_shape=jax.ShapeDtypeStruct((m_out, c_out), jnp.float32),
    )(g4, s3, wa, wb, bias, gamma, beta, pool_m, exp_m)


def _dec2_stage(g3, s2, wa, wb, block_rows, out_rows=None):
    m = g3.shape[0]
    c_out = wa.shape[1]
    m_out = m if out_rows is None else out_rows

    def body(g_ref, s_ref, wa_ref, wb_ref, o_ref):
        o_ref[...] = (jnp.dot(g_ref[...], wa_ref[...],
                              preferred_element_type=jnp.float32)
                      + jnp.dot(s_ref[...], wb_ref[...],
                                preferred_element_type=jnp.float32))

    specs = [
        pl.BlockSpec((block_rows, g3.shape[1]), lambda i: (i, 0)),
        pl.BlockSpec((block_rows, s2.shape[1]), lambda i: (i, 0)),
        pl.BlockSpec(wa.shape, lambda i: (0, 0)),
        pl.BlockSpec(wb.shape, lambda i: (0, 0)),
    ]
    return pl.pallas_call(
        body,
        grid=(m // block_rows,),
        in_specs=specs,
        out_specs=pl.BlockSpec((block_rows, c_out), lambda i: (i, 0)),
        out_shape=jax.ShapeDtypeStruct((m_out, c_out), jnp.float32),
    )(g3, s2, wa, wb)


def _sc_pool_colsplit(table, idx_flat, m_pad, k_nb, c_dim, r_blk, nbuf):
    c_half = c_dim // 2
    chunk = m_pad // _NS
    assert chunk % r_blk == 0
    nblk = chunk // r_blk
    n_idx = r_blk * k_nb
    assert n_idx % 8 == 0
    scale = 1.0 / k_nb
    nch = c_half // 16
    n_rows = table.shape[0]
    nsl = n_rows // _NS

    splits = []
    off = 0
    while off < n_idx:
        sz = min(128, n_idx - off)
        splits.append((off, sz))
        off += sz

    mesh = plsc.VectorSubcoreMesh(core_axis_name="c", subcore_axis_name="s")

    scratch = [pltpu.VMEM((n_idx,), jnp.int32) for _ in range(nbuf)]
    scratch += [pltpu.VMEM((n_idx, c_half), jnp.float32) for _ in range(nbuf)]
    scratch.append(pltpu.VMEM((r_blk, c_half), jnp.float32))
    scratch.append(pltpu.VMEM_SHARED((n_rows, c_half), jnp.float32))
    scratch += [pltpu.SemaphoreType.DMA for _ in range(nbuf)]

    def body(table_hbm, idx_hbm, out_hbm, *scr):
        idxb = scr[:nbuf]
        rowsb = scr[nbuf:2 * nbuf]
        out_v = scr[2 * nbuf]
        tbl_s = scr[2 * nbuf + 1]
        sems = scr[2 * nbuf + 2:]

        sid = lax.axis_index("s")
        cid = lax.axis_index("c")
        base = sid * chunk
        cb = cid * c_half

        pltpu.sync_copy(table_hbm.at[pl.ds(sid * nsl, nsl), pl.ds(cb, c_half)],
                        tbl_s.at[pl.ds(sid * nsl, nsl)])
        plsc.subcore_barrier()

        def issue(b, p):
            rb = base + b * r_blk
            pltpu.sync_copy(idx_hbm.at[pl.ds(rb * k_nb, n_idx)], idxb[p])
            for (o, s) in splits:
                pltpu.async_copy(tbl_s.at[idxb[p].at[pl.ds(o, s)]],
                                 rowsb[p].at[pl.ds(o, s)], sems[p])

        def drain(p):
            for (o, s) in splits:
                pltpu.make_async_copy(tbl_s.at[idxb[p].at[pl.ds(o, s)]],
                                      rowsb[p].at[pl.ds(o, s)],
                                      sems[p]).wait()

        def finish(b, p):
            rb = base + b * r_blk
            rows_v = rowsb[p]

            def row(r, rc):
                jb = r * k_nb
                accs = [None] * nch
                for kk in range(k_nb):
                    for ci in range(nch):
                        v = rows_v[jb + kk, pl.ds(ci * 16, 16)]
                        accs[ci] = v if kk == 0 else accs[ci] + v
                for ci in range(nch):
                    out_v[r, pl.ds(ci * 16, 16)] = accs[ci] * scale
                return rc

            lax.fori_loop(0, r_blk, row, 0)
            pltpu.sync_copy(out_v,
                            out_hbm.at[pl.ds(rb, r_blk), pl.ds(cb, c_half)])

        issue(0, 0)

        def blk(b, carry):
            @pl.when(b % 2 == 0)
            def _():
                pl.when(b + 1 < nblk)(lambda: issue(b + 1, 1))
                drain(0)
                finish(b, 0)

            @pl.when(b % 2 == 1)
            def _():
                pl.when(b + 1 < nblk)(lambda: issue(b + 1, 0))
                drain(1)
                finish(b, 1)

            return carry
        lax.fori_loop(0, nblk, blk, 0)

    f = pl.kernel(
        body,
        out_type=jax.ShapeDtypeStruct((m_pad, c_dim), jnp.float32),
        mesh=mesh,
        compiler_params=pltpu.CompilerParams(use_tc_tiling_on_sc=False),
        scratch_types=scratch,
    )
    return f(table, idx_flat)


def _sc_pool_bf16(table, idx_flat, m_pad, k_nb, c_dim, r_blk, nbuf,
                  col_split, skew=None):
    cw = c_dim // 2 if col_split else c_dim
    chunk = m_pad // (_NS if col_split else _NW)
    if skew is not None:
        assert not col_split and _NS * (skew[0] + skew[1]) == m_pad
        assert skew[0] % r_blk == 0 and skew[1] % r_blk == 0
        chunk = max(skew)
    assert chunk % r_blk == 0
    n_idx = r_blk * k_nb
    assert n_idx % 8 == 0
    scale = 1.0 / k_nb
    nch2 = cw // 32
    n_rows = table.shape[0]
    nsl = n_rows // _NS

    splits = []
    off = 0
    while off < n_idx:
        sz = min(128, n_idx - off)
        splits.append((off, sz))
        off += sz

    mesh = plsc.VectorSubcoreMesh(core_axis_name="c", subcore_axis_name="s")

    scratch = [pltpu.VMEM((chunk * k_nb,), jnp.int32)]
    scratch += [pltpu.VMEM((n_idx, cw), jnp.bfloat16) for _ in range(nbuf)]
    scratch.append(pltpu.VMEM((r_blk, cw), jnp.float32))
    if col_split:
        scratch.append(pltpu.VMEM_SHARED((n_rows, cw), jnp.bfloat16))
    scratch += [pltpu.SemaphoreType.DMA for _ in range(nbuf)]

    def body(table_hbm, idx_hbm, out_hbm, *scr):
        pos = 0
        idx_v = scr[pos]; pos += 1
        rowsb = scr[pos:pos + nbuf]; pos += nbuf
        out_v = scr[pos]; pos += 1
        tbl_s = None
        if col_split:
            tbl_s = scr[pos]; pos += 1
        sems = scr[pos:]

        sid = lax.axis_index("s")
        cid = lax.axis_index("c")
        if col_split:
            cb = cid * cw
            pltpu.sync_copy(
                table_hbm.at[pl.ds(sid * nsl, nsl), pl.ds(cb, cw)],
                tbl_s.at[pl.ds(sid * nsl, nsl)])
            plsc.subcore_barrier()
            src = tbl_s
        else:
            cb = 0
            src = table_hbm

        def issue(b, p):
            for (o, s) in splits:
                pltpu.async_copy(
                    src.at[idx_v.at[pl.ds(b * n_idx + o, s)]],
                    rowsb[p].at[pl.ds(o, s)], sems[p])

        def drain(b, p):
            for (o, s) in splits:
                pltpu.make_async_copy(
                    src.at[idx_v.at[pl.ds(b * n_idx + o, s)]],
                    rowsb[p].at[pl.ds(o, s)], sems[p]).wait()

        def finish(base, b, p):
            rb = base + b * r_blk
            rows_v = rowsb[p]

            def row(r, rc):
                jb = r * k_nb
                accs = [None] * (2 * nch2)
                for kk in range(k_nb):
                    for ci in range(nch2):
                        vb = rows_v[jb + kk, pl.ds(ci * 32, 32)]
                        a, b2 = plsc.unpack(
                            vb, format=plsc.PackFormat.INTERLEAVED,
                            preferred_element_type=jnp.float32)
                        if kk == 0:
                            accs[2 * ci], accs[2 * ci + 1] = a, b2
                        else:
                            accs[2 * ci] = accs[2 * ci] + a
                            accs[2 * ci + 1] = accs[2 * ci + 1] + b2
                for ci in range(2 * nch2):
                    out_v[r, pl.ds(ci * 16, 16)] = accs[ci] * scale
                return rc

            lax.fori_loop(0, r_blk, row, 0)
            if col_split:
                pltpu.sync_copy(
                    out_v, out_hbm.at[pl.ds(rb, r_blk), pl.ds(cb, cw)])
            else:
                pltpu.sync_copy(out_v, out_hbm.at[pl.ds(rb, r_blk)])

        def run(base, my_chunk):
            pltpu.sync_copy(idx_hbm.at[pl.ds(base * k_nb, my_chunk * k_nb)],
                            idx_v.at[pl.ds(0, my_chunk * k_nb)])
            my_nblk = my_chunk // r_blk
            issue(0, 0)

            def blk(b, carry):
                @pl.when(b % 2 == 0)
                def _():
                    pl.when(b + 1 < my_nblk)(lambda: issue(b + 1, 1))
                    drain(b, 0)
                    finish(base, b, 0)

                @pl.when(b % 2 == 1)
                def _():
                    pl.when(b + 1 < my_nblk)(lambda: issue(b + 1, 0))
                    drain(b, 1)
                    finish(base, b, 1)

                return carry
            lax.fori_loop(0, my_nblk, blk, 0)

        if col_split:
            run(sid * chunk, chunk)
        elif skew is None:
            run((sid * _NC + cid) * chunk, chunk)
        else:
            ca, cbn = skew
            pl.when(cid == 0)(lambda: run(sid * ca, ca))
            pl.when(cid == 1)(lambda: run(_NS * ca + sid * cbn, cbn))

    f = pl.kernel(
        body,
        out_type=jax.ShapeDtypeStruct((m_pad, c_dim), jnp.float32),
        mesh=mesh,
        compiler_params=pltpu.CompilerParams(use_tc_tiling_on_sc=False,
                                             needs_layout_passes=False),
        scratch_types=scratch,
    )
    return f(table, idx_flat)


def _deint_perm(c_dim):
    p = jnp.arange(c_dim, dtype=jnp.int32)
    chunk = (p // 32) * 32
    pos = p % 32
    return jnp.where(pos < 16, chunk + 2 * pos, chunk + 2 * (pos - 16) + 1)


def _sc_pool(table, idx_flat, m_pad, k_nb, c_dim, r_blk, nbuf,
             use_spmem=False, tc_tiling=False, stage_split=None):
    chunk = m_pad // _NW
    assert chunk % r_blk == 0 and chunk % 8 == 0
    nblk = chunk // r_blk
    n_idx = r_blk * k_nb
    assert n_idx % 8 == 0
    scale = 1.0 / k_nb
    nch = c_dim // 16

    splits = []
    off = 0
    while off < n_idx:
        sz = min(128, n_idx - off)
        splits.append((off, sz))
        off += sz

    mesh = plsc.VectorSubcoreMesh(core_axis_name="c", subcore_axis_name="s")

    n_rows = table.shape[0]
    scratch = [pltpu.VMEM((n_idx,), jnp.int32) for _ in range(nbuf)]
    scratch += [pltpu.VMEM((n_idx, c_dim), jnp.float32) for _ in range(nbuf)]
    if k_nb > 1:
        scratch.append(pltpu.VMEM((r_blk, c_dim), jnp.float32))
    if use_spmem:
        scratch.append(pltpu.VMEM_SHARED((n_rows, c_dim), jnp.float32))
    scratch += [pltpu.SemaphoreType.DMA for _ in range(nbuf)]

    def body(table_hbm, idx_hbm, out_hbm, *scr):
        pos = 0
        idxb = scr[pos:pos + nbuf]; pos += nbuf
        rowsb = scr[pos:pos + nbuf]; pos += nbuf
        out_v = None
        if k_nb > 1:
            out_v = scr[pos]; pos += 1
        tbl_s = None
        if use_spmem:
            tbl_s = scr[pos]; pos += 1
        sems = scr[pos:]

        wid = lax.axis_index("s") * _NC + lax.axis_index("c")
        base = wid * chunk

        if use_spmem:
            nsl, nstage = (stage_split if stage_split is not None
                           else (n_rows // _NS, _NS))
            sid = lax.axis_index("s")
            pl.when(sid < nstage)(lambda: pltpu.sync_copy(
                table_hbm.at[pl.ds(sid * nsl, nsl)],
                tbl_s.at[pl.ds(sid * nsl, nsl)]))
            plsc.subcore_barrier()
            src = tbl_s
        else:
            src = table_hbm

        def issue(b, p):
            rb = base + b * r_blk
            pltpu.sync_copy(idx_hbm.at[pl.ds(rb * k_nb, n_idx)], idxb[p])
            for (o, s) in splits:
                pltpu.async_copy(src.at[idxb[p].at[pl.ds(o, s)]],
                                 rowsb[p].at[pl.ds(o, s)], sems[p])

        def drain(p):
            for (o, s) in splits:
                pltpu.make_async_copy(src.at[idxb[p].at[pl.ds(o, s)]],
                                      rowsb[p].at[pl.ds(o, s)],
                                      sems[p]).wait()

        def finish(b, p):
            rb = base + b * r_blk
            rows_v = rowsb[p]
            if k_nb == 1:
                pltpu.sync_copy(rows_v, out_hbm.at[pl.ds(rb, r_blk)])
                return

            def row(r, rc):
                jb = r * k_nb
                accs = [None] * nch
                for kk in range(k_nb):
                    for ci in range(nch):
                        v = rows_v[jb + kk, pl.ds(ci * 16, 16)]
                        accs[ci] = v if kk == 0 else accs[ci] + v
                for ci in range(nch):
                    out_v[r, pl.ds(ci * 16, 16)] = accs[ci] * scale
                return rc

            lax.fori_loop(0, r_blk, row, 0)
            pltpu.sync_copy(out_v, out_hbm.at[pl.ds(rb, r_blk)])

        if nbuf == 1:
            def blk(b, carry):
                issue(b, 0)
                drain(0)
                finish(b, 0)
                return carry
            lax.fori_loop(0, nblk, blk, 0)
        else:
            issue(0, 0)

            def blk(b, carry):
                @pl.when(b % 2 == 0)
                def _():
                    pl.when(b + 1 < nblk)(lambda: issue(b + 1, 1))
                    drain(0)
                    finish(b, 0)

                @pl.when(b % 2 == 1)
                def _():
                    pl.when(b + 1 < nblk)(lambda: issue(b + 1, 0))
                    drain(1)
                    finish(b, 1)

                return carry
            lax.fori_loop(0, nblk, blk, 0)

    f = pl.kernel(
        body,
        out_type=jax.ShapeDtypeStruct((m_pad, c_dim), jnp.float32),
        mesh=mesh,
        compiler_params=pltpu.CompilerParams(use_tc_tiling_on_sc=tc_tiling),
        scratch_types=scratch,
    )
    return f(table, idx_flat)


def _pad_rows(a, m_pad):
    return jnp.pad(a, ((0, m_pad - a.shape[0]),) + ((0, 0),) * (a.ndim - 1))


def kernel(feats, points_s1, points_s2, points_s3, points_s4,
           W0, W1, W2, W3, W4, W5, W6, W7, W8, W9, W10,
           W_d3, b_d3, gamma_d3, beta_d3, W_d2,
           subsampling_0, subsampling_1, subsampling_2,
           upsampling_1, upsampling_2):
    m2p, m3p, m4p = 12544, 3328, 1024

    sub0 = _pad_rows(subsampling_0, m2p).reshape(-1)
    sub1 = _pad_rows(subsampling_1, m3p).reshape(-1)
    sub2 = _pad_rows(subsampling_2, m4p).reshape(-1)
    up1 = _pad_rows(upsampling_1, m2p).reshape(-1)
    up2 = _pad_rows(upsampling_2, m3p).reshape(-1)

    ch = jnp.arange(256, dtype=jnp.int32)
    gr = jnp.arange(_GROUPS, dtype=jnp.int32)
    pool_m = ((ch[:, None] // 8) == gr[None, :]).astype(jnp.float32) / 8.0
    exp_m = ((gr[:, None] == ch[None, :] // 8)).astype(jnp.float32)

    s1 = _mlp_stage(feats, [W0, W1], [False, False], 2000,
                    out_dtype=jnp.bfloat16)
    p0 = _sc_pool_bf16(s1, sub0, m2p, 26, 64, 56, 2,
                       col_split=False, skew=(448, 336))
    s2, s2b = _mlp_stage(p0, [W2[_deint_perm(64)], W3, W4],
                         [False, False, True], 1568,
                         extra_bf16=True)
    p1 = _sc_pool_bf16(s2b, sub1, m3p, 31, 128, 16, 2,
                       col_split=True)
    s3, s3b = _mlp_stage(p1, [W5[_deint_perm(128)], W6, W7],
                         [False, False, True], 832,
                         extra_bf16=True)
    p2 = _sc_pool_bf16(s3b, sub2, m4p, 33, 256, 8, 2,
                       col_split=True)
    s4 = _mlp_stage(p2, [W8[_deint_perm(256)], W9, W10],
                    [False, False, True], 1024,
                    out_rows=782)

    g4 = _sc_pool(s4, up2, m3p, 1, 512, 104, 1, tc_tiling=True)
    l3 = _dec3_stage(g4, s3, W_d3[:512], W_d3[512:],
                     b_d3.reshape(1, -1), gamma_d3.reshape(1, -1),
                     beta_d3.reshape(1, -1), pool_m, exp_m, 832,
                     out_rows=3125)
    g3 = _sc_pool(l3, up1, m2p, 1, 256, 392, 1, tc_tiling=True)
    l2 = _dec2_stage(g3, s2, W_d2[:256], W_d2[256:], 1568,
                     out_rows=12500)

    return (l2, l3, s4)
